# baseline (speedup 1.0000x reference)
import functools

import jax
import jax.numpy as jnp
from jax import lax
from jax.experimental import pallas as pl
from jax.experimental.pallas import tpu as pltpu
from jax.experimental.pallas import tpu_sc as plsc

_N = 8192
_C = 256
_MIDS = 32
_S16 = 16
_S8 = 8
_BN = 256
_GRID = _N // _BN
_EPS = 1e-5
_CH = 128
_NW = 32
_PER_W = (_N * _S16) // _NW
_NCH = _PER_W // _CH


def _f32(shape):
    return jax.ShapeDtypeStruct(shape, jnp.float32)


def _qkv_body(x_ref, wq, bq, wk, bk, wv, bv, xq_o, xk_o, xv_o):
    x = x_ref[...]
    xq_o[...] = jnp.dot(x, wq[...], preferred_element_type=jnp.float32) + bq[...]
    xk_o[...] = jnp.dot(x, wk[...], preferred_element_type=jnp.float32) + bk[...]
    xv_o[...] = jnp.dot(x, wv[...], preferred_element_type=jnp.float32) + bv[...]


def _qkv(x, wq, bq, wk, bk, wv, bv):
    full = pl.BlockSpec((_C, _C), lambda i: (0, 0))
    vec = pl.BlockSpec((1, _C), lambda i: (0, 0))
    blk = pl.BlockSpec((_BN, _C), lambda i: (i, 0))
    return pl.pallas_call(
        _qkv_body,
        grid=(_GRID,),
        in_specs=[blk, full, vec, full, vec, full, vec],
        out_specs=[blk, blk, blk],
        out_shape=[_f32((_N, _C))] * 3,
    )(x, wq, bq, wk, bk, wv, bv)


_NG = 64
_GW = _N // _NG
_NR = 6
_KBN = 128


def _knn_body(p_ref, pt_ref, idx_ref, d_ref):
    pb = p_ref[...]
    pt = pt_ref[...]
    sqq = jnp.sum(pb * pb, axis=1, keepdims=True)
    sqc = jnp.sum(pt * pt, axis=0, keepdims=True)
    d = (sqq + sqc
         - 2.0 * jnp.dot(pb, pt, preferred_element_type=jnp.float32))
    d_ref[...] = d.reshape(_KBN, _NG, _GW)
    lanein = lax.broadcasted_iota(jnp.int32, (_KBN, _NG, _GW), 2)
    gbase = _GW * lax.broadcasted_iota(jnp.int32, (_KBN, _NG), 1)
    col16 = lax.broadcasted_iota(jnp.int32, (_KBN, _S16), 1)
    inf = jnp.float32(jnp.inf)
    big = jnp.int32(2 ** 30)

    pv, pi = [], []
    for _ in range(_NR):
        dd = d_ref[...]
        gm = jnp.min(dd, axis=2)
        hit = dd == gm[:, :, None]
        jarg = jnp.min(jnp.where(hit, lanein, big), axis=2)
        d_ref[...] = jnp.where(lanein == jarg[:, :, None], inf, dd)
        pv.append(gm)
        pi.append(gbase + jarg)
    pv = jnp.concatenate(pv, axis=1)
    pi = jnp.concatenate(pi, axis=1)

    out = jnp.zeros((_KBN, _S16), jnp.int32)
    for s in range(_S16):
        m = jnp.min(pv, axis=1, keepdims=True)
        j = jnp.min(jnp.where(pv == m, pi, big), axis=1, keepdims=True)
        out = jnp.where(col16 == s, j, out)
        pv = jnp.where(pi == j, inf, pv)

    og = lax.div(out, jnp.int32(_GW))
    g64 = lax.broadcasted_iota(jnp.int32, (1, 1, _NG), 2)
    cnt = jnp.sum((og[:, :, None] == g64).astype(jnp.int32), axis=1)
    bad = jnp.max(cnt) >= _NR

    @pl.when(jnp.logical_not(bad))
    def _():
        idx_ref[...] = out

    @pl.when(bad)
    def _():
        d_ref[...] = d.reshape(_KBN, _NG, _GW)
        gidx = gbase[:, :, None] + lanein
        o2 = jnp.zeros((_KBN, _S16), jnp.int32)
        for s in range(_S16):
            dd = d_ref[...]
            m2 = jnp.min(dd, axis=2)
            m = jnp.min(m2, axis=1, keepdims=True)
            hit = dd == m[:, :, None]
            j2 = jnp.min(jnp.where(hit, gidx, big), axis=2)
            j = jnp.min(j2, axis=1, keepdims=True)
            o2 = jnp.where(col16 == s, j, o2)
            d_ref[...] = jnp.where(gidx == j[:, :, None], inf, dd)
        idx_ref[...] = o2


def _knn(p, pt):
    return pl.pallas_call(
        _knn_body,
        grid=(_N // _KBN,),
        in_specs=[pl.BlockSpec((_KBN, 3), lambda i: (i, 0)),
                  pl.BlockSpec((3, _N), lambda i: (0, 0))],
        out_specs=pl.BlockSpec((_KBN, _S16), lambda i: (i, 0)),
        out_shape=jax.ShapeDtypeStruct((_N, _S16), jnp.int32),
        scratch_shapes=[pltpu.VMEM((_KBN, _NG, _GW), jnp.float32)],
    )(p, pt)


_TW = 2 * _C + 128


def _sc_gather(tab, idx):
    mesh = plsc.VectorSubcoreMesh(core_axis_name="c", subcore_axis_name="s")

    @functools.partial(
        pl.kernel, mesh=mesh,
        out_type=_f32((_N * _S16, _TW)),
        scratch_types=[pltpu.VMEM((_CH,), jnp.int32),
                       pltpu.VMEM((_CH, _TW), jnp.float32),
                       pltpu.SemaphoreType.DMA],
    )
    def gather_kernel(t_hbm, idx_hbm, g_hbm, idxb, rb, sem):
        wid = lax.axis_index("s") * 2 + lax.axis_index("c")
        base = wid * _PER_W

        def chunk(c, carry):
            off = base + c * _CH
            pltpu.sync_copy(idx_hbm.at[pl.ds(off, _CH)], idxb)
            pltpu.async_copy(t_hbm.at[idxb], rb, sem).wait()
            pltpu.sync_copy(rb, g_hbm.at[pl.ds(off, _CH)])
            return carry

        lax.fori_loop(0, _NCH, chunk, 0)

    return gather_kernel(tab, idx)


def _pos_mlp(gpp, pb, wp1, bp1, hst, gpg, bpn, wp2, bp2, ns, row):
    pr0 = gpp[:, :ns, 0:1] - pb[:, None, 0:1]
    pr1 = gpp[:, :ns, 1:2] - pb[:, None, 1:2]
    pr2 = gpp[:, :ns, 2:3] - pb[:, None, 2:3]
    h = (pr0 * wp1[0:1, :][None] + pr1 * wp1[1:2, :][None]
         + pr2 * wp1[2:3, :][None] + bp1[None])
    cnt = float(_N * ns)
    mean = hst[row:row + 1, :] / cnt
    var = hst[row + 1:row + 2, :] / cnt - mean * mean
    scale = lax.rsqrt(var + _EPS) * gpg
    hb = (h - mean[None]) * scale[None] + bpn[None]
    r = jnp.maximum(hb, 0.0)
    return (r[..., 0:1] * wp2[0:1, :][None]
            + r[..., 1:2] * wp2[1:2, :][None]
            + r[..., 2:3] * wp2[2:3, :][None] + bp2[None])


def _sum2(a):
    return jnp.sum(a.reshape(-1, a.shape[-1]), axis=0, keepdims=True)


def _softmax1(z):
    m = jnp.max(z, axis=1, keepdims=True)
    e = jnp.exp(z - m)
    return e / jnp.sum(e, axis=1, keepdims=True)


def _hstats_body(gp_ref, p_ref, wp1, bp1, st_ref):
    i = pl.program_id(0)
    gpp = gp_ref[...]
    pb = p_ref[...]
    w = wp1[...]
    h = ((gpp[..., 0:1] - pb[:, None, 0:1]) * w[0:1, :][None]
         + (gpp[..., 1:2] - pb[:, None, 1:2]) * w[1:2, :][None]
         + (gpp[..., 2:3] - pb[:, None, 2:3]) * w[2:3, :][None]
         + bp1[...][None])
    h2 = h * h
    st = jnp.concatenate(
        [_sum2(h), _sum2(h2), _sum2(h[:, :_S8]), _sum2(h2[:, :_S8])], axis=0)

    @pl.when(i == 0)
    def _():
        st_ref[...] = jnp.zeros_like(st_ref)

    st_ref[...] += st


def _hstats(gc3, p, wp1, bp1):
    return pl.pallas_call(
        _hstats_body,
        grid=(_GRID,),
        in_specs=[pl.BlockSpec((_BN, _S16, 128), lambda i: (i, 0, 4)),
                  pl.BlockSpec((_BN, 3), lambda i: (i, 0)),
                  pl.BlockSpec((3, 3), lambda i: (0, 0)),
                  pl.BlockSpec((1, 3), lambda i: (0, 0))],
        out_specs=pl.BlockSpec((4, 3), lambda i: (0, 0)),
        out_shape=_f32((4, 3)),
    )(gc3, p, wp1, bp1)


def _wstats_body(gk_ref, gp_ref, p_ref, xq_ref, wp1, bp1, gpg, bpn, wp2, bp2,
                 hst, ws_ref):
    i = pl.program_id(0)
    gp = gp_ref[...]
    p16 = p_ref[...]
    gk = gk_ref[...]
    xq = xq_ref[...]
    prt16 = _pos_mlp(gp, p16, wp1[...], bp1[...], hst[...], gpg[...],
                     bpn[...], wp2[...], bp2[...], _S16, 0)
    prt8 = _pos_mlp(gp, p16, wp1[...], bp1[...], hst[...], gpg[...],
                    bpn[...], wp2[...], bp2[...], _S8, 2)
    w1 = gk - xq[:, None, :] + prt16
    w2 = gk[:, :_S8] - xq[:, None, :] + prt8
    st = jnp.concatenate(
        [_sum2(w1), _sum2(w1 * w1), _sum2(w2), _sum2(w2 * w2)], axis=0)

    @pl.when(i == 0)
    def _():
        ws_ref[...] = jnp.zeros_like(ws_ref)

    ws_ref[...] += st


def _wstats(gc3, p, xq, wp1, bp1, gpg, bpn, wp2, bp2, hst):
    return pl.pallas_call(
        _wstats_body,
        grid=(_GRID,),
        in_specs=[pl.BlockSpec((_BN, _S16, _C), lambda i: (i, 0, 0)),
                  pl.BlockSpec((_BN, _S16, 128), lambda i: (i, 0, 4)),
                  pl.BlockSpec((_BN, 3), lambda i: (i, 0)),
                  pl.BlockSpec((_BN, _C), lambda i: (i, 0)),
                  pl.BlockSpec((3, 3), lambda i: (0, 0)),
                  pl.BlockSpec((1, 3), lambda i: (0, 0)),
                  pl.BlockSpec((1, 3), lambda i: (0, 0)),
                  pl.BlockSpec((1, 3), lambda i: (0, 0)),
                  pl.BlockSpec((3, _C), lambda i: (0, 0)),
                  pl.BlockSpec((1, _C), lambda i: (0, 0)),
                  pl.BlockSpec((4, 3), lambda i: (0, 0))],
        out_specs=pl.BlockSpec((4, _C), lambda i: (0, 0)),
        out_shape=_f32((4, _C)),
    )(gc3, gc3, p, xq, wp1, bp1, gpg, bpn, wp2, bp2, hst)


def _s3_body(gk_ref, gp_ref, p_ref, xq_ref, wp1, bp1, gpg, bpn, wp2, bp2,
             hst, ws, gw1, ww1, bw1, s31_ref, s32_ref, st_ref):
    i = pl.program_id(0)
    gp = gp_ref[...]
    p16 = p_ref[...]
    gk = gk_ref[...]
    xq = xq_ref[...]
    prt16 = _pos_mlp(gp, p16, wp1[...], bp1[...], hst[...], gpg[...],
                     bpn[...], wp2[...], bp2[...], _S16, 0)
    prt8 = _pos_mlp(gp, p16, wp1[...], bp1[...], hst[...], gpg[...],
                    bpn[...], wp2[...], bp2[...], _S8, 2)
    wsv = ws[...]

    def branch(w, ns, row, out_ref):
        cnt = float(_N * ns)
        mean = wsv[row:row + 1, :] / cnt
        var = wsv[row + 1:row + 2, :] / cnt - mean * mean
        c1 = (gw1[...] * lax.rsqrt(var + _EPS))[None]
        a = _softmax1(w * c1)
        a = _softmax1(a)
        s3 = jnp.dot(a.reshape(_BN * ns, _C), ww1[...],
                     preferred_element_type=jnp.float32) + bw1[...]
        s3 = _softmax1(s3.reshape(_BN, ns, _MIDS))
        out_ref[...] = s3
        return jnp.concatenate([_sum2(s3), _sum2(s3 * s3)], axis=0)

    w1 = gk - xq[:, None, :] + prt16
    w2 = gk[:, :_S8] - xq[:, None, :] + prt8
    st1 = branch(w1, _S16, 0, s31_ref)
    st2 = branch(w2, _S8, 2, s32_ref)
    st = jnp.concatenate([st1, st2], axis=0)

    @pl.when(i == 0)
    def _():
        st_ref[...] = jnp.zeros_like(st_ref)

    st_ref[...] += st


def _s3_stage(gc3, p, xq, wp1, bp1, gpg, bpn, wp2, bp2, hst, ws,
              gw1, ww1, bw1):
    return pl.pallas_call(
        _s3_body,
        grid=(_GRID,),
        in_specs=[pl.BlockSpec((_BN, _S16, _C), lambda i: (i, 0, 0)),
                  pl.BlockSpec((_BN, _S16, 128), lambda i: (i, 0, 4)),
                  pl.BlockSpec((_BN, 3), lambda i: (i, 0)),
                  pl.BlockSpec((_BN, _C), lambda i: (i, 0)),
                  pl.BlockSpec((3, 3), lambda i: (0, 0)),
                  pl.BlockSpec((1, 3), lambda i: (0, 0)),
                  pl.BlockSpec((1, 3), lambda i: (0, 0)),
                  pl.BlockSpec((1, 3), lambda i: (0, 0)),
                  pl.BlockSpec((3, _C), lambda i: (0, 0)),
                  pl.BlockSpec((1, _C), lambda i: (0, 0)),
                  pl.BlockSpec((4, 3), lambda i: (0, 0)),
                  pl.BlockSpec((4, _C), lambda i: (0, 0)),
                  pl.BlockSpec((1, _C), lambda i: (0, 0)),
                  pl.BlockSpec((_C, _MIDS), lambda i: (0, 0)),
                  pl.BlockSpec((1, _MIDS), lambda i: (0, 0))],
        out_specs=[pl.BlockSpec((_BN, _S16, _MIDS), lambda i: (i, 0, 0)),
                   pl.BlockSpec((_BN, _S8, _MIDS), lambda i: (i, 0, 0)),
                   pl.BlockSpec((4, _MIDS), lambda i: (0, 0))],
        out_shape=[_f32((_N, _S16, _MIDS)), _f32((_N, _S8, _MIDS)),
                   _f32((4, _MIDS))],
    )(gc3, gc3, p, xq, wp1, bp1, gpg, bpn, wp2, bp2, hst, ws, gw1, ww1, bw1)


def _final_body(gv_ref, gp_ref, p_ref, s31_ref, s32_ref, wp1, bp1, gpg, bpn,
                wp2, bp2, hst, s3st, gw2, ww2, bw2, x1_ref, x2_ref):
    gp = gp_ref[...]
    p16 = p_ref[...]
    gv = gv_ref[...]
    stv = s3st[...]
    prt16 = _pos_mlp(gp, p16, wp1[...], bp1[...], hst[...], gpg[...],
                     bpn[...], wp2[...], bp2[...], _S16, 0)
    prt8 = _pos_mlp(gp, p16, wp1[...], bp1[...], hst[...], gpg[...],
                    bpn[...], wp2[...], bp2[...], _S8, 2)

    def branch(s3, vp, ns, row, out_ref):
        cnt = float(_N * ns)
        mean = stv[row:row + 1, :] / cnt
        var = stv[row + 1:row + 2, :] / cnt - mean * mean
        c2 = (gw2[...] * lax.rsqrt(var + _EPS))[None]
        b = _softmax1(s3 * c2)
        b = _softmax1(b)
        wf = jnp.dot(b.reshape(_BN * ns, _MIDS), ww2[...],
                     preferred_element_type=jnp.float32) + bw2[...]
        wf = _softmax1(wf.reshape(_BN, ns, _MIDS))
        wt = jnp.concatenate([wf] * (_C // _MIDS), axis=2)
        out_ref[...] = jnp.sum(vp * wt, axis=1)

    branch(s31_ref[...], gv + prt16, _S16, 0, x1_ref)
    branch(s32_ref[...], gv[:, :_S8] + prt8, _S8, 2, x2_ref)


def _final_stage(gc3, p, s31, s32, wp1, bp1, gpg, bpn, wp2, bp2,
                 hst, s3st, gw2, ww2, bw2):
    return pl.pallas_call(
        _final_body,
        grid=(_GRID,),
        in_specs=[pl.BlockSpec((_BN, _S16, _C), lambda i: (i, 0, 1)),
                  pl.BlockSpec((_BN, _S16, 128), lambda i: (i, 0, 4)),
                  pl.BlockSpec((_BN, 3), lambda i: (i, 0)),
                  pl.BlockSpec((_BN, _S16, _MIDS), lambda i: (i, 0, 0)),
                  pl.BlockSpec((_BN, _S8, _MIDS), lambda i: (i, 0, 0)),
                  pl.BlockSpec((3, 3), lambda i: (0, 0)),
                  pl.BlockSpec((1, 3), lambda i: (0, 0)),
                  pl.BlockSpec((1, 3), lambda i: (0, 0)),
                  pl.BlockSpec((1, 3), lambda i: (0, 0)),
                  pl.BlockSpec((3, _C), lambda i: (0, 0)),
                  pl.BlockSpec((1, _C), lambda i: (0, 0)),
                  pl.BlockSpec((4, 3), lambda i: (0, 0)),
                  pl.BlockSpec((4, _MIDS), lambda i: (0, 0)),
                  pl.BlockSpec((1, _MIDS), lambda i: (0, 0)),
                  pl.BlockSpec((_MIDS, _MIDS), lambda i: (0, 0)),
                  pl.BlockSpec((1, _MIDS), lambda i: (0, 0))],
        out_specs=[pl.BlockSpec((_BN, _C), lambda i: (i, 0)),
                   pl.BlockSpec((_BN, _C), lambda i: (i, 0))],
        out_shape=[_f32((_N, _C)), _f32((_N, _C))],
    )(gc3, gc3, p, s31, s32, wp1, bp1, gpg, bpn, wp2, bp2, hst, s3st,
      gw2, ww2, bw2)


def kernel(p, x, o, Wq, bq, Wk, bk, Wv, bv, Wp1, bp1, gp, bpn, Wp2, bp2,
           gw1, bw1n, Ww1, bw1, gw2, bw2n, Ww2, bw2):
    del o, bw1n, bw2n
    r = lambda v: v.reshape(1, -1)
    p128 = jnp.concatenate([p, jnp.zeros((_N, 125), jnp.float32)], axis=1)
    xq, xk, xv = _qkv(x, Wq, r(bq), Wk, r(bk), Wv, r(bv))
    tab = jnp.concatenate([xk, xv, p128], axis=1)
    idx = _knn(p, p.T)
    gc = _sc_gather(tab, idx.reshape(-1))
    gc3 = gc.reshape(_N, _S16, _TW)
    hst = _hstats(gc3, p, Wp1, r(bp1))
    ws = _wstats(gc3, p, xq, Wp1, r(bp1), r(gp), r(bpn), Wp2, r(bp2), hst)
    s31, s32, s3st = _s3_stage(gc3, p, xq, Wp1, r(bp1), r(gp), r(bpn),
                               Wp2, r(bp2), hst, ws, r(gw1), Ww1, r(bw1))
    x1, x2 = _final_stage(gc3, p, s31, s32, Wp1, r(bp1), r(gp), r(bpn),
                          Wp2, r(bp2), hst, s3st, r(gw2), Ww2, r(bw2))
    return jnp.concatenate([x1, x2], axis=-1)


# speedup vs baseline: 1.9311x; 1.9311x over previous
import functools

import jax
import jax.numpy as jnp
from jax import lax
from jax.experimental import pallas as pl
from jax.experimental.pallas import tpu as pltpu
from jax.experimental.pallas import tpu_sc as plsc

_N = 8192
_C = 256
_MIDS = 32
_S16 = 16
_S8 = 8
_BN = 256
_GRID = _N // _BN
_EPS = 1e-5
_CH = 128
_NW = 32
_PER_W = (_N * _S16) // _NW
_NCH = _PER_W // _CH


def _f32(shape):
    return jax.ShapeDtypeStruct(shape, jnp.float32)


def _qkv_body(x_ref, wq, bq, wk, bk, wv, bv, xq_o, xk_o, xv_o):
    x = x_ref[...]
    xq_o[...] = jnp.dot(x, wq[...], preferred_element_type=jnp.float32) + bq[...]
    xk_o[...] = jnp.dot(x, wk[...], preferred_element_type=jnp.float32) + bk[...]
    xv_o[...] = jnp.dot(x, wv[...], preferred_element_type=jnp.float32) + bv[...]


def _qkv(x, wq, bq, wk, bk, wv, bv):
    full = pl.BlockSpec((_C, _C), lambda i: (0, 0))
    vec = pl.BlockSpec((1, _C), lambda i: (0, 0))
    blk = pl.BlockSpec((_BN, _C), lambda i: (i, 0))
    return pl.pallas_call(
        _qkv_body,
        grid=(_GRID,),
        in_specs=[blk, full, vec, full, vec, full, vec],
        out_specs=[blk, blk, blk],
        out_shape=[_f32((_N, _C))] * 3,
    )(x, wq, bq, wk, bk, wv, bv)


_NG = 128
_GD = _N // _NG
_NR = 5
_KBN = 128


def _knn_body(p_ref, pt_ref, idx_ref, d_ref):
    pb = p_ref[...]
    pt = pt_ref[...]
    sqq = jnp.sum(pb * pb, axis=1, keepdims=True)
    sqc = jnp.sum(pt * pt, axis=0, keepdims=True)
    d = (sqq + sqc
         - 2.0 * jnp.dot(pb, pt, preferred_element_type=jnp.float32))
    d_ref[...] = d.reshape(_KBN, _GD, _NG)
    aio = lax.broadcasted_iota(jnp.int32, (_KBN, _GD, _NG), 1)
    bio = lax.broadcasted_iota(jnp.int32, (_KBN, _NG), 1)
    col16 = lax.broadcasted_iota(jnp.int32, (_KBN, _S16), 1)
    inf = jnp.float32(jnp.inf)
    big = jnp.int32(2 ** 30)

    pv, pi = [], []
    for _ in range(_NR):
        dd = d_ref[...]
        gm = jnp.min(dd, axis=1)
        hit = dd == gm[:, None, :]
        jarg = jnp.min(jnp.where(hit, aio, big), axis=1)
        d_ref[...] = jnp.where(aio == jarg[:, None, :], inf, dd)
        pv.append(gm)
        pi.append(jarg * _NG + bio)
    pv = jnp.concatenate(pv, axis=1)
    pi = jnp.concatenate(pi, axis=1)

    out = jnp.zeros((_KBN, _S16), jnp.int32)
    for s in range(_S16):
        m = jnp.min(pv, axis=1, keepdims=True)
        j = jnp.min(jnp.where(pv == m, pi, big), axis=1, keepdims=True)
        out = jnp.where(col16 == s, j, out)
        pv = jnp.where(pi == j, inf, pv)

    og = lax.rem(out, jnp.int32(_NG))
    gng = lax.broadcasted_iota(jnp.int32, (1, 1, _NG), 2)
    cnt = jnp.sum((og[:, :, None] == gng).astype(jnp.int32), axis=1)
    bad = jnp.max(cnt) >= _NR

    @pl.when(jnp.logical_not(bad))
    def _():
        idx_ref[...] = out

    @pl.when(bad)
    def _():
        d_ref[...] = d.reshape(_KBN, _GD, _NG)
        gidx = aio * _NG + bio[:, None, :]
        o2 = jnp.zeros((_KBN, _S16), jnp.int32)
        for s in range(_S16):
            dd = d_ref[...]
            m2 = jnp.min(dd, axis=1)
            m = jnp.min(m2, axis=1, keepdims=True)
            hit = dd == m[:, None, :]
            j2 = jnp.min(jnp.where(hit, gidx, big), axis=1)
            j = jnp.min(j2, axis=1, keepdims=True)
            o2 = jnp.where(col16 == s, j, o2)
            d_ref[...] = jnp.where(gidx == j[:, None, :], inf, dd)
        idx_ref[...] = o2


def _knn(p, pt):
    return pl.pallas_call(
        _knn_body,
        grid=(_N // _KBN,),
        in_specs=[pl.BlockSpec((_KBN, 3), lambda i: (i, 0)),
                  pl.BlockSpec((3, _N), lambda i: (0, 0))],
        out_specs=pl.BlockSpec((_KBN, _S16), lambda i: (i, 0)),
        out_shape=jax.ShapeDtypeStruct((_N, _S16), jnp.int32),
        scratch_shapes=[pltpu.VMEM((_KBN, _GD, _NG), jnp.float32)],
    )(p, pt)


_TW = 2 * _C + 128


def _sc_gather(tab, idx):
    mesh = plsc.VectorSubcoreMesh(core_axis_name="c", subcore_axis_name="s")

    @functools.partial(
        pl.kernel, mesh=mesh,
        out_type=_f32((_N * _S16, _TW)),
        scratch_types=[pltpu.VMEM((_CH,), jnp.int32),
                       pltpu.VMEM((_CH, _TW), jnp.float32),
                       pltpu.SemaphoreType.DMA],
    )
    def gather_kernel(t_hbm, idx_hbm, g_hbm, idxb, rb, sem):
        wid = lax.axis_index("s") * 2 + lax.axis_index("c")
        base = wid * _PER_W

        def chunk(c, carry):
            off = base + c * _CH
            pltpu.sync_copy(idx_hbm.at[pl.ds(off, _CH)], idxb)
            pltpu.async_copy(t_hbm.at[idxb], rb, sem).wait()
            pltpu.sync_copy(rb, g_hbm.at[pl.ds(off, _CH)])
            return carry

        lax.fori_loop(0, _NCH, chunk, 0)

    return gather_kernel(tab, idx)


def _pos_mlp(gpp, pb, wp1, bp1, hst, gpg, bpn, wp2, bp2, ns, row):
    pr0 = gpp[:, :ns, 0:1] - pb[:, None, 0:1]
    pr1 = gpp[:, :ns, 1:2] - pb[:, None, 1:2]
    pr2 = gpp[:, :ns, 2:3] - pb[:, None, 2:3]
    h = (pr0 * wp1[0:1, :][None] + pr1 * wp1[1:2, :][None]
         + pr2 * wp1[2:3, :][None] + bp1[None])
    cnt = float(_N * ns)
    mean = hst[row:row + 1, :] / cnt
    var = hst[row + 1:row + 2, :] / cnt - mean * mean
    scale = lax.rsqrt(var + _EPS) * gpg
    hb = (h - mean[None]) * scale[None] + bpn[None]
    r = jnp.maximum(hb, 0.0)
    return (r[..., 0:1] * wp2[0:1, :][None]
            + r[..., 1:2] * wp2[1:2, :][None]
            + r[..., 2:3] * wp2[2:3, :][None] + bp2[None])


def _sum2(a):
    return jnp.sum(a.reshape(-1, a.shape[-1]), axis=0, keepdims=True)


def _softmax1(z):
    m = jnp.max(z, axis=1, keepdims=True)
    e = jnp.exp(z - m)
    return e / jnp.sum(e, axis=1, keepdims=True)


def _hstats_body(gp_ref, p_ref, wp1, bp1, st_ref):
    i = pl.program_id(0)
    gpp = gp_ref[...]
    pb = p_ref[...]
    w = wp1[...]
    h = ((gpp[..., 0:1] - pb[:, None, 0:1]) * w[0:1, :][None]
         + (gpp[..., 1:2] - pb[:, None, 1:2]) * w[1:2, :][None]
         + (gpp[..., 2:3] - pb[:, None, 2:3]) * w[2:3, :][None]
         + bp1[...][None])
    h2 = h * h
    st = jnp.concatenate(
        [_sum2(h), _sum2(h2), _sum2(h[:, :_S8]), _sum2(h2[:, :_S8])], axis=0)

    @pl.when(i == 0)
    def _():
        st_ref[...] = jnp.zeros_like(st_ref)

    st_ref[...] += st


def _hstats(gc3, p, wp1, bp1):
    return pl.pallas_call(
        _hstats_body,
        grid=(_GRID,),
        in_specs=[pl.BlockSpec((_BN, _S16, 128), lambda i: (i, 0, 4)),
                  pl.BlockSpec((_BN, 3), lambda i: (i, 0)),
                  pl.BlockSpec((3, 3), lambda i: (0, 0)),
                  pl.BlockSpec((1, 3), lambda i: (0, 0))],
        out_specs=pl.BlockSpec((4, 3), lambda i: (0, 0)),
        out_shape=_f32((4, 3)),
    )(gc3, p, wp1, bp1)


def _wstats_body(gk_ref, gp_ref, p_ref, xq_ref, wp1, bp1, gpg, bpn, wp2, bp2,
                 hst, ws_ref):
    i = pl.program_id(0)
    gp = gp_ref[...]
    p16 = p_ref[...]
    gk = gk_ref[...]
    xq = xq_ref[...]
    prt16 = _pos_mlp(gp, p16, wp1[...], bp1[...], hst[...], gpg[...],
                     bpn[...], wp2[...], bp2[...], _S16, 0)
    prt8 = _pos_mlp(gp, p16, wp1[...], bp1[...], hst[...], gpg[...],
                    bpn[...], wp2[...], bp2[...], _S8, 2)
    w1 = gk - xq[:, None, :] + prt16
    w2 = gk[:, :_S8] - xq[:, None, :] + prt8
    st = jnp.concatenate(
        [_sum2(w1), _sum2(w1 * w1), _sum2(w2), _sum2(w2 * w2)], axis=0)

    @pl.when(i == 0)
    def _():
        ws_ref[...] = jnp.zeros_like(ws_ref)

    ws_ref[...] += st


def _wstats(gc3, p, xq, wp1, bp1, gpg, bpn, wp2, bp2, hst):
    return pl.pallas_call(
        _wstats_body,
        grid=(_GRID,),
        in_specs=[pl.BlockSpec((_BN, _S16, _C), lambda i: (i, 0, 0)),
                  pl.BlockSpec((_BN, _S16, 128), lambda i: (i, 0, 4)),
                  pl.BlockSpec((_BN, 3), lambda i: (i, 0)),
                  pl.BlockSpec((_BN, _C), lambda i: (i, 0)),
                  pl.BlockSpec((3, 3), lambda i: (0, 0)),
                  pl.BlockSpec((1, 3), lambda i: (0, 0)),
                  pl.BlockSpec((1, 3), lambda i: (0, 0)),
                  pl.BlockSpec((1, 3), lambda i: (0, 0)),
                  pl.BlockSpec((3, _C), lambda i: (0, 0)),
                  pl.BlockSpec((1, _C), lambda i: (0, 0)),
                  pl.BlockSpec((4, 3), lambda i: (0, 0))],
        out_specs=pl.BlockSpec((4, _C), lambda i: (0, 0)),
        out_shape=_f32((4, _C)),
    )(gc3, gc3, p, xq, wp1, bp1, gpg, bpn, wp2, bp2, hst)


def _s3_body(gk_ref, gp_ref, p_ref, xq_ref, wp1, bp1, gpg, bpn, wp2, bp2,
             hst, ws, gw1, ww1, bw1, s31_ref, s32_ref, st_ref):
    i = pl.program_id(0)
    gp = gp_ref[...]
    p16 = p_ref[...]
    gk = gk_ref[...]
    xq = xq_ref[...]
    prt16 = _pos_mlp(gp, p16, wp1[...], bp1[...], hst[...], gpg[...],
                     bpn[...], wp2[...], bp2[...], _S16, 0)
    prt8 = _pos_mlp(gp, p16, wp1[...], bp1[...], hst[...], gpg[...],
                    bpn[...], wp2[...], bp2[...], _S8, 2)
    wsv = ws[...]

    def branch(w, ns, row, out_ref):
        cnt = float(_N * ns)
        mean = wsv[row:row + 1, :] / cnt
        var = wsv[row + 1:row + 2, :] / cnt - mean * mean
        c1 = (gw1[...] * lax.rsqrt(var + _EPS))[None]
        a = _softmax1(w * c1)
        a = _softmax1(a)
        s3 = jnp.dot(a.reshape(_BN * ns, _C), ww1[...],
                     preferred_element_type=jnp.float32) + bw1[...]
        s3 = _softmax1(s3.reshape(_BN, ns, _MIDS))
        out_ref[...] = s3
        return jnp.concatenate([_sum2(s3), _sum2(s3 * s3)], axis=0)

    w1 = gk - xq[:, None, :] + prt16
    w2 = gk[:, :_S8] - xq[:, None, :] + prt8
    st1 = branch(w1, _S16, 0, s31_ref)
    st2 = branch(w2, _S8, 2, s32_ref)
    st = jnp.concatenate([st1, st2], axis=0)

    @pl.when(i == 0)
    def _():
        st_ref[...] = jnp.zeros_like(st_ref)

    st_ref[...] += st


def _s3_stage(gc3, p, xq, wp1, bp1, gpg, bpn, wp2, bp2, hst, ws,
              gw1, ww1, bw1):
    return pl.pallas_call(
        _s3_body,
        grid=(_GRID,),
        in_specs=[pl.BlockSpec((_BN, _S16, _C), lambda i: (i, 0, 0)),
                  pl.BlockSpec((_BN, _S16, 128), lambda i: (i, 0, 4)),
                  pl.BlockSpec((_BN, 3), lambda i: (i, 0)),
                  pl.BlockSpec((_BN, _C), lambda i: (i, 0)),
                  pl.BlockSpec((3, 3), lambda i: (0, 0)),
                  pl.BlockSpec((1, 3), lambda i: (0, 0)),
                  pl.BlockSpec((1, 3), lambda i: (0, 0)),
                  pl.BlockSpec((1, 3), lambda i: (0, 0)),
                  pl.BlockSpec((3, _C), lambda i: (0, 0)),
                  pl.BlockSpec((1, _C), lambda i: (0, 0)),
                  pl.BlockSpec((4, 3), lambda i: (0, 0)),
                  pl.BlockSpec((4, _C), lambda i: (0, 0)),
                  pl.BlockSpec((1, _C), lambda i: (0, 0)),
                  pl.BlockSpec((_C, _MIDS), lambda i: (0, 0)),
                  pl.BlockSpec((1, _MIDS), lambda i: (0, 0))],
        out_specs=[pl.BlockSpec((_BN, _S16, _MIDS), lambda i: (i, 0, 0)),
                   pl.BlockSpec((_BN, _S8, _MIDS), lambda i: (i, 0, 0)),
                   pl.BlockSpec((4, _MIDS), lambda i: (0, 0))],
        out_shape=[_f32((_N, _S16, _MIDS)), _f32((_N, _S8, _MIDS)),
                   _f32((4, _MIDS))],
    )(gc3, gc3, p, xq, wp1, bp1, gpg, bpn, wp2, bp2, hst, ws, gw1, ww1, bw1)


def _final_body(gv_ref, gp_ref, p_ref, s31_ref, s32_ref, wp1, bp1, gpg, bpn,
                wp2, bp2, hst, s3st, gw2, ww2, bw2, x1_ref, x2_ref):
    gp = gp_ref[...]
    p16 = p_ref[...]
    gv = gv_ref[...]
    stv = s3st[...]
    prt16 = _pos_mlp(gp, p16, wp1[...], bp1[...], hst[...], gpg[...],
                     bpn[...], wp2[...], bp2[...], _S16, 0)
    prt8 = _pos_mlp(gp, p16, wp1[...], bp1[...], hst[...], gpg[...],
                    bpn[...], wp2[...], bp2[...], _S8, 2)

    def branch(s3, vp, ns, row, out_ref):
        cnt = float(_N * ns)
        mean = stv[row:row + 1, :] / cnt
        var = stv[row + 1:row + 2, :] / cnt - mean * mean
        c2 = (gw2[...] * lax.rsqrt(var + _EPS))[None]
        b = _softmax1(s3 * c2)
        b = _softmax1(b)
        wf = jnp.dot(b.reshape(_BN * ns, _MIDS), ww2[...],
                     preferred_element_type=jnp.float32) + bw2[...]
        wf = _softmax1(wf.reshape(_BN, ns, _MIDS))
        wt = jnp.concatenate([wf] * (_C // _MIDS), axis=2)
        out_ref[...] = jnp.sum(vp * wt, axis=1)

    branch(s31_ref[...], gv + prt16, _S16, 0, x1_ref)
    branch(s32_ref[...], gv[:, :_S8] + prt8, _S8, 2, x2_ref)


def _final_stage(gc3, p, s31, s32, wp1, bp1, gpg, bpn, wp2, bp2,
                 hst, s3st, gw2, ww2, bw2):
    return pl.pallas_call(
        _final_body,
        grid=(_GRID,),
        in_specs=[pl.BlockSpec((_BN, _S16, _C), lambda i: (i, 0, 1)),
                  pl.BlockSpec((_BN, _S16, 128), lambda i: (i, 0, 4)),
                  pl.BlockSpec((_BN, 3), lambda i: (i, 0)),
                  pl.BlockSpec((_BN, _S16, _MIDS), lambda i: (i, 0, 0)),
                  pl.BlockSpec((_BN, _S8, _MIDS), lambda i: (i, 0, 0)),
                  pl.BlockSpec((3, 3), lambda i: (0, 0)),
                  pl.BlockSpec((1, 3), lambda i: (0, 0)),
                  pl.BlockSpec((1, 3), lambda i: (0, 0)),
                  pl.BlockSpec((1, 3), lambda i: (0, 0)),
                  pl.BlockSpec((3, _C), lambda i: (0, 0)),
                  pl.BlockSpec((1, _C), lambda i: (0, 0)),
                  pl.BlockSpec((4, 3), lambda i: (0, 0)),
                  pl.BlockSpec((4, _MIDS), lambda i: (0, 0)),
                  pl.BlockSpec((1, _MIDS), lambda i: (0, 0)),
                  pl.BlockSpec((_MIDS, _MIDS), lambda i: (0, 0)),
                  pl.BlockSpec((1, _MIDS), lambda i: (0, 0))],
        out_specs=[pl.BlockSpec((_BN, _C), lambda i: (i, 0)),
                   pl.BlockSpec((_BN, _C), lambda i: (i, 0))],
        out_shape=[_f32((_N, _C)), _f32((_N, _C))],
    )(gc3, gc3, p, s31, s32, wp1, bp1, gpg, bpn, wp2, bp2, hst, s3st,
      gw2, ww2, bw2)


def kernel(p, x, o, Wq, bq, Wk, bk, Wv, bv, Wp1, bp1, gp, bpn, Wp2, bp2,
           gw1, bw1n, Ww1, bw1, gw2, bw2n, Ww2, bw2):
    del o, bw1n, bw2n
    r = lambda v: v.reshape(1, -1)
    p128 = jnp.concatenate([p, jnp.zeros((_N, 125), jnp.float32)], axis=1)
    xq, xk, xv = _qkv(x, Wq, r(bq), Wk, r(bk), Wv, r(bv))
    tab = jnp.concatenate([xk, xv, p128], axis=1)
    idx = _knn(p, p.T)
    gc = _sc_gather(tab, idx.reshape(-1))
    gc3 = gc.reshape(_N, _S16, _TW)
    hst = _hstats(gc3, p, Wp1, r(bp1))
    ws = _wstats(gc3, p, xq, Wp1, r(bp1), r(gp), r(bpn), Wp2, r(bp2), hst)
    s31, s32, s3st = _s3_stage(gc3, p, xq, Wp1, r(bp1), r(gp), r(bpn),
                               Wp2, r(bp2), hst, ws, r(gw1), Ww1, r(bw1))
    x1, x2 = _final_stage(gc3, p, s31, s32, Wp1, r(bp1), r(gp), r(bpn),
                          Wp2, r(bp2), hst, s3st, r(gw2), Ww2, r(bw2))
    return jnp.concatenate([x1, x2], axis=-1)


# speedup vs baseline: 1.9469x; 1.0082x over previous
import functools

import jax
import jax.numpy as jnp
from jax import lax
from jax.experimental import pallas as pl
from jax.experimental.pallas import tpu as pltpu
from jax.experimental.pallas import tpu_sc as plsc

_N = 8192
_C = 256
_MIDS = 32
_S16 = 16
_S8 = 8
_BN = 256
_GRID = _N // _BN
_EPS = 1e-5
_CH = 64
_NW = 32
_PER_W = (_N * _S16) // _NW
_NCH = _PER_W // _CH


def _f32(shape):
    return jax.ShapeDtypeStruct(shape, jnp.float32)


def _qkv_body(x_ref, wq, bq, wk, bk, wv, bv, xq_o, xk_o, xv_o):
    x = x_ref[...]
    xq_o[...] = jnp.dot(x, wq[...], preferred_element_type=jnp.float32) + bq[...]
    xk_o[...] = jnp.dot(x, wk[...], preferred_element_type=jnp.float32) + bk[...]
    xv_o[...] = jnp.dot(x, wv[...], preferred_element_type=jnp.float32) + bv[...]


def _qkv(x, wq, bq, wk, bk, wv, bv):
    full = pl.BlockSpec((_C, _C), lambda i: (0, 0))
    vec = pl.BlockSpec((1, _C), lambda i: (0, 0))
    blk = pl.BlockSpec((_BN, _C), lambda i: (i, 0))
    return pl.pallas_call(
        _qkv_body,
        grid=(_GRID,),
        in_specs=[blk, full, vec, full, vec, full, vec],
        out_specs=[blk, blk, blk],
        out_shape=[_f32((_N, _C))] * 3,
    )(x, wq, bq, wk, bk, wv, bv)


_NG = 128
_GD = _N // _NG
_NR = 5
_KBN = 128


def _knn_body(p_ref, pt_ref, idx_ref, d_ref):
    pb = p_ref[...]
    pt = pt_ref[...]
    sqq = jnp.sum(pb * pb, axis=1, keepdims=True)
    sqc = jnp.sum(pt * pt, axis=0, keepdims=True)
    d = (sqq + sqc
         - 2.0 * jnp.dot(pb, pt, preferred_element_type=jnp.float32))
    d_ref[...] = d.reshape(_KBN, _GD, _NG)
    aio = lax.broadcasted_iota(jnp.int32, (_KBN, _GD, _NG), 1)
    bio = lax.broadcasted_iota(jnp.int32, (_KBN, _NG), 1)
    col16 = lax.broadcasted_iota(jnp.int32, (_KBN, _S16), 1)
    inf = jnp.float32(jnp.inf)
    big = jnp.int32(2 ** 30)

    pv, pi = [], []
    for _ in range(_NR):
        dd = d_ref[...]
        gm = jnp.min(dd, axis=1)
        hit = dd == gm[:, None, :]
        jarg = jnp.min(jnp.where(hit, aio, big), axis=1)
        d_ref[...] = jnp.where(aio == jarg[:, None, :], inf, dd)
        pv.append(gm)
        pi.append(jarg * _NG + bio)
    pv = jnp.concatenate(pv, axis=1)
    pi = jnp.concatenate(pi, axis=1)

    out = jnp.zeros((_KBN, _S16), jnp.int32)
    for s in range(_S16):
        m = jnp.min(pv, axis=1, keepdims=True)
        j = jnp.min(jnp.where(pv == m, pi, big), axis=1, keepdims=True)
        out = jnp.where(col16 == s, j, out)
        pv = jnp.where(pi == j, inf, pv)

    og = lax.rem(out, jnp.int32(_NG))
    gng = lax.broadcasted_iota(jnp.int32, (1, 1, _NG), 2)
    cnt = jnp.sum((og[:, :, None] == gng).astype(jnp.int32), axis=1)
    bad = jnp.max(cnt) >= _NR

    @pl.when(jnp.logical_not(bad))
    def _():
        idx_ref[...] = out

    @pl.when(bad)
    def _():
        d_ref[...] = d.reshape(_KBN, _GD, _NG)
        gidx = aio * _NG + bio[:, None, :]
        o2 = jnp.zeros((_KBN, _S16), jnp.int32)
        for s in range(_S16):
            dd = d_ref[...]
            m2 = jnp.min(dd, axis=1)
            m = jnp.min(m2, axis=1, keepdims=True)
            hit = dd == m[:, None, :]
            j2 = jnp.min(jnp.where(hit, gidx, big), axis=1)
            j = jnp.min(j2, axis=1, keepdims=True)
            o2 = jnp.where(col16 == s, j, o2)
            d_ref[...] = jnp.where(gidx == j[:, None, :], inf, dd)
        idx_ref[...] = o2


def _knn(p, pt):
    return pl.pallas_call(
        _knn_body,
        grid=(_N // _KBN,),
        in_specs=[pl.BlockSpec((_KBN, 3), lambda i: (i, 0)),
                  pl.BlockSpec((3, _N), lambda i: (0, 0))],
        out_specs=pl.BlockSpec((_KBN, _S16), lambda i: (i, 0)),
        out_shape=jax.ShapeDtypeStruct((_N, _S16), jnp.int32),
        scratch_shapes=[pltpu.VMEM((_KBN, _GD, _NG), jnp.float32)],
    )(p, pt)


_TW = 2 * _C + 128


def _sc_gather(tab, idx):
    mesh = plsc.VectorSubcoreMesh(core_axis_name="c", subcore_axis_name="s")

    @functools.partial(
        pl.kernel, mesh=mesh,
        out_type=_f32((_N * _S16, _TW)),
        scratch_types=[pltpu.VMEM((_CH,), jnp.int32),
                       pltpu.VMEM((_CH,), jnp.int32),
                       pltpu.VMEM((_CH, _TW), jnp.float32),
                       pltpu.VMEM((_CH, _TW), jnp.float32),
                       pltpu.SemaphoreType.DMA,
                       pltpu.SemaphoreType.DMA,
                       pltpu.SemaphoreType.DMA,
                       pltpu.SemaphoreType.DMA],
    )
    def gather_kernel(t_hbm, idx_hbm, g_hbm, idxb0, idxb1, rb0, rb1,
                      gsem0, gsem1, wsem0, wsem1):
        wid = lax.axis_index("s") * 2 + lax.axis_index("c")
        base = wid * _PER_W
        idxb = (idxb0, idxb1)
        rb = (rb0, rb1)
        gsem = (gsem0, gsem1)
        wsem = (wsem0, wsem1)

        def fire(c, b):
            off = base + c * _CH
            pltpu.sync_copy(idx_hbm.at[pl.ds(off, _CH)], idxb[b])
            pltpu.async_copy(t_hbm.at[idxb[b]], rb[b], gsem[b])

        def step(c, carry):
            def per_buf(bb):
                off = base + c * _CH
                pltpu.make_async_copy(t_hbm.at[idxb[bb]], rb[bb],
                                      gsem[bb]).wait()
                pltpu.async_copy(rb[bb], g_hbm.at[pl.ds(off, _CH)], wsem[bb])
                pltpu.make_async_copy(rb[bb], g_hbm.at[pl.ds(off, _CH)],
                                      wsem[bb]).wait()

                @pl.when(c < _NCH - 2)
                def _():
                    fire(c + 2, bb)

            b = lax.rem(c, 2)

            @pl.when(b == 0)
            def _():
                per_buf(0)

            @pl.when(b == 1)
            def _():
                per_buf(1)

            return carry

        fire(0, 0)
        fire(1, 1)
        lax.fori_loop(0, _NCH, step, 0)

    return gather_kernel(tab, idx)


def _pos_mlp(gpp, pb, wp1, bp1, hst, gpg, bpn, wp2, bp2, ns, row):
    pr0 = gpp[:, :ns, 0:1] - pb[:, None, 0:1]
    pr1 = gpp[:, :ns, 1:2] - pb[:, None, 1:2]
    pr2 = gpp[:, :ns, 2:3] - pb[:, None, 2:3]
    h = (pr0 * wp1[0:1, :][None] + pr1 * wp1[1:2, :][None]
         + pr2 * wp1[2:3, :][None] + bp1[None])
    cnt = float(_N * ns)
    mean = hst[row:row + 1, :] / cnt
    var = hst[row + 1:row + 2, :] / cnt - mean * mean
    scale = lax.rsqrt(var + _EPS) * gpg
    hb = (h - mean[None]) * scale[None] + bpn[None]
    r = jnp.maximum(hb, 0.0)
    return (r[..., 0:1] * wp2[0:1, :][None]
            + r[..., 1:2] * wp2[1:2, :][None]
            + r[..., 2:3] * wp2[2:3, :][None] + bp2[None])


def _sum2(a):
    return jnp.sum(a.reshape(-1, a.shape[-1]), axis=0, keepdims=True)


def _softmax1(z):
    m = jnp.max(z, axis=1, keepdims=True)
    e = jnp.exp(z - m)
    return e / jnp.sum(e, axis=1, keepdims=True)


def _hstats_body(gp_ref, p_ref, wp1, bp1, st_ref):
    i = pl.program_id(0)
    gpp = gp_ref[...]
    pb = p_ref[...]
    w = wp1[...]
    h = ((gpp[..., 0:1] - pb[:, None, 0:1]) * w[0:1, :][None]
         + (gpp[..., 1:2] - pb[:, None, 1:2]) * w[1:2, :][None]
         + (gpp[..., 2:3] - pb[:, None, 2:3]) * w[2:3, :][None]
         + bp1[...][None])
    h2 = h * h
    st = jnp.concatenate(
        [_sum2(h), _sum2(h2), _sum2(h[:, :_S8]), _sum2(h2[:, :_S8])], axis=0)

    @pl.when(i == 0)
    def _():
        st_ref[...] = jnp.zeros_like(st_ref)

    st_ref[...] += st


def _hstats(gc3, p, wp1, bp1):
    return pl.pallas_call(
        _hstats_body,
        grid=(_GRID,),
        in_specs=[pl.BlockSpec((_BN, _S16, 128), lambda i: (i, 0, 4)),
                  pl.BlockSpec((_BN, 3), lambda i: (i, 0)),
                  pl.BlockSpec((3, 3), lambda i: (0, 0)),
                  pl.BlockSpec((1, 3), lambda i: (0, 0))],
        out_specs=pl.BlockSpec((4, 3), lambda i: (0, 0)),
        out_shape=_f32((4, 3)),
    )(gc3, p, wp1, bp1)


def _wstats_body(gk_ref, gp_ref, p_ref, xq_ref, wp1, bp1, gpg, bpn, wp2, bp2,
                 hst, ws_ref):
    i = pl.program_id(0)
    gp = gp_ref[...]
    p16 = p_ref[...]
    gk = gk_ref[...]
    xq = xq_ref[...]
    prt16 = _pos_mlp(gp, p16, wp1[...], bp1[...], hst[...], gpg[...],
                     bpn[...], wp2[...], bp2[...], _S16, 0)
    prt8 = _pos_mlp(gp, p16, wp1[...], bp1[...], hst[...], gpg[...],
                    bpn[...], wp2[...], bp2[...], _S8, 2)
    w1 = gk - xq[:, None, :] + prt16
    w2 = gk[:, :_S8] - xq[:, None, :] + prt8
    st = jnp.concatenate(
        [_sum2(w1), _sum2(w1 * w1), _sum2(w2), _sum2(w2 * w2)], axis=0)

    @pl.when(i == 0)
    def _():
        ws_ref[...] = jnp.zeros_like(ws_ref)

    ws_ref[...] += st


def _wstats(gc3, p, xq, wp1, bp1, gpg, bpn, wp2, bp2, hst):
    return pl.pallas_call(
        _wstats_body,
        grid=(_GRID,),
        in_specs=[pl.BlockSpec((_BN, _S16, _C), lambda i: (i, 0, 0)),
                  pl.BlockSpec((_BN, _S16, 128), lambda i: (i, 0, 4)),
                  pl.BlockSpec((_BN, 3), lambda i: (i, 0)),
                  pl.BlockSpec((_BN, _C), lambda i: (i, 0)),
                  pl.BlockSpec((3, 3), lambda i: (0, 0)),
                  pl.BlockSpec((1, 3), lambda i: (0, 0)),
                  pl.BlockSpec((1, 3), lambda i: (0, 0)),
                  pl.BlockSpec((1, 3), lambda i: (0, 0)),
                  pl.BlockSpec((3, _C), lambda i: (0, 0)),
                  pl.BlockSpec((1, _C), lambda i: (0, 0)),
                  pl.BlockSpec((4, 3), lambda i: (0, 0))],
        out_specs=pl.BlockSpec((4, _C), lambda i: (0, 0)),
        out_shape=_f32((4, _C)),
    )(gc3, gc3, p, xq, wp1, bp1, gpg, bpn, wp2, bp2, hst)


def _s3_body(gk_ref, gp_ref, p_ref, xq_ref, wp1, bp1, gpg, bpn, wp2, bp2,
             hst, ws, gw1, ww1, bw1, s31_ref, s32_ref, st_ref):
    i = pl.program_id(0)
    gp = gp_ref[...]
    p16 = p_ref[...]
    gk = gk_ref[...]
    xq = xq_ref[...]
    prt16 = _pos_mlp(gp, p16, wp1[...], bp1[...], hst[...], gpg[...],
                     bpn[...], wp2[...], bp2[...], _S16, 0)
    prt8 = _pos_mlp(gp, p16, wp1[...], bp1[...], hst[...], gpg[...],
                    bpn[...], wp2[...], bp2[...], _S8, 2)
    wsv = ws[...]

    def branch(w, ns, row, out_ref):
        cnt = float(_N * ns)
        mean = wsv[row:row + 1, :] / cnt
        var = wsv[row + 1:row + 2, :] / cnt - mean * mean
        c1 = (gw1[...] * lax.rsqrt(var + _EPS))[None]
        a = _softmax1(w * c1)
        a = _softmax1(a)
        s3 = jnp.dot(a.reshape(_BN * ns, _C), ww1[...],
                     preferred_element_type=jnp.float32) + bw1[...]
        s3 = _softmax1(s3.reshape(_BN, ns, _MIDS))
        out_ref[...] = s3
        return jnp.concatenate([_sum2(s3), _sum2(s3 * s3)], axis=0)

    w1 = gk - xq[:, None, :] + prt16
    w2 = gk[:, :_S8] - xq[:, None, :] + prt8
    st1 = branch(w1, _S16, 0, s31_ref)
    st2 = branch(w2, _S8, 2, s32_ref)
    st = jnp.concatenate([st1, st2], axis=0)

    @pl.when(i == 0)
    def _():
        st_ref[...] = jnp.zeros_like(st_ref)

    st_ref[...] += st


def _s3_stage(gc3, p, xq, wp1, bp1, gpg, bpn, wp2, bp2, hst, ws,
              gw1, ww1, bw1):
    return pl.pallas_call(
        _s3_body,
        grid=(_GRID,),
        in_specs=[pl.BlockSpec((_BN, _S16, _C), lambda i: (i, 0, 0)),
                  pl.BlockSpec((_BN, _S16, 128), lambda i: (i, 0, 4)),
                  pl.BlockSpec((_BN, 3), lambda i: (i, 0)),
                  pl.BlockSpec((_BN, _C), lambda i: (i, 0)),
                  pl.BlockSpec((3, 3), lambda i: (0, 0)),
                  pl.BlockSpec((1, 3), lambda i: (0, 0)),
                  pl.BlockSpec((1, 3), lambda i: (0, 0)),
                  pl.BlockSpec((1, 3), lambda i: (0, 0)),
                  pl.BlockSpec((3, _C), lambda i: (0, 0)),
                  pl.BlockSpec((1, _C), lambda i: (0, 0)),
                  pl.BlockSpec((4, 3), lambda i: (0, 0)),
                  pl.BlockSpec((4, _C), lambda i: (0, 0)),
                  pl.BlockSpec((1, _C), lambda i: (0, 0)),
                  pl.BlockSpec((_C, _MIDS), lambda i: (0, 0)),
                  pl.BlockSpec((1, _MIDS), lambda i: (0, 0))],
        out_specs=[pl.BlockSpec((_BN, _S16, _MIDS), lambda i: (i, 0, 0)),
                   pl.BlockSpec((_BN, _S8, _MIDS), lambda i: (i, 0, 0)),
                   pl.BlockSpec((4, _MIDS), lambda i: (0, 0))],
        out_shape=[_f32((_N, _S16, _MIDS)), _f32((_N, _S8, _MIDS)),
                   _f32((4, _MIDS))],
    )(gc3, gc3, p, xq, wp1, bp1, gpg, bpn, wp2, bp2, hst, ws, gw1, ww1, bw1)


def _final_body(gv_ref, gp_ref, p_ref, s31_ref, s32_ref, wp1, bp1, gpg, bpn,
                wp2, bp2, hst, s3st, gw2, ww2, bw2, x1_ref, x2_ref):
    gp = gp_ref[...]
    p16 = p_ref[...]
    gv = gv_ref[...]
    stv = s3st[...]
    prt16 = _pos_mlp(gp, p16, wp1[...], bp1[...], hst[...], gpg[...],
                     bpn[...], wp2[...], bp2[...], _S16, 0)
    prt8 = _pos_mlp(gp, p16, wp1[...], bp1[...], hst[...], gpg[...],
                    bpn[...], wp2[...], bp2[...], _S8, 2)

    def branch(s3, vp, ns, row, out_ref):
        cnt = float(_N * ns)
        mean = stv[row:row + 1, :] / cnt
        var = stv[row + 1:row + 2, :] / cnt - mean * mean
        c2 = (gw2[...] * lax.rsqrt(var + _EPS))[None]
        b = _softmax1(s3 * c2)
        b = _softmax1(b)
        wf = jnp.dot(b.reshape(_BN * ns, _MIDS), ww2[...],
                     preferred_element_type=jnp.float32) + bw2[...]
        wf = _softmax1(wf.reshape(_BN, ns, _MIDS))
        wt = jnp.concatenate([wf] * (_C // _MIDS), axis=2)
        out_ref[...] = jnp.sum(vp * wt, axis=1)

    branch(s31_ref[...], gv + prt16, _S16, 0, x1_ref)
    branch(s32_ref[...], gv[:, :_S8] + prt8, _S8, 2, x2_ref)


def _final_stage(gc3, p, s31, s32, wp1, bp1, gpg, bpn, wp2, bp2,
                 hst, s3st, gw2, ww2, bw2):
    return pl.pallas_call(
        _final_body,
        grid=(_GRID,),
        in_specs=[pl.BlockSpec((_BN, _S16, _C), lambda i: (i, 0, 1)),
                  pl.BlockSpec((_BN, _S16, 128), lambda i: (i, 0, 4)),
                  pl.BlockSpec((_BN, 3), lambda i: (i, 0)),
                  pl.BlockSpec((_BN, _S16, _MIDS), lambda i: (i, 0, 0)),
                  pl.BlockSpec((_BN, _S8, _MIDS), lambda i: (i, 0, 0)),
                  pl.BlockSpec((3, 3), lambda i: (0, 0)),
                  pl.BlockSpec((1, 3), lambda i: (0, 0)),
                  pl.BlockSpec((1, 3), lambda i: (0, 0)),
                  pl.BlockSpec((1, 3), lambda i: (0, 0)),
                  pl.BlockSpec((3, _C), lambda i: (0, 0)),
                  pl.BlockSpec((1, _C), lambda i: (0, 0)),
                  pl.BlockSpec((4, 3), lambda i: (0, 0)),
                  pl.BlockSpec((4, _MIDS), lambda i: (0, 0)),
                  pl.BlockSpec((1, _MIDS), lambda i: (0, 0)),
                  pl.BlockSpec((_MIDS, _MIDS), lambda i: (0, 0)),
                  pl.BlockSpec((1, _MIDS), lambda i: (0, 0))],
        out_specs=[pl.BlockSpec((_BN, _C), lambda i: (i, 0)),
                   pl.BlockSpec((_BN, _C), lambda i: (i, 0))],
        out_shape=[_f32((_N, _C)), _f32((_N, _C))],
    )(gc3, gc3, p, s31, s32, wp1, bp1, gpg, bpn, wp2, bp2, hst, s3st,
      gw2, ww2, bw2)


def kernel(p, x, o, Wq, bq, Wk, bk, Wv, bv, Wp1, bp1, gp, bpn, Wp2, bp2,
           gw1, bw1n, Ww1, bw1, gw2, bw2n, Ww2, bw2):
    del o, bw1n, bw2n
    r = lambda v: v.reshape(1, -1)
    p128 = jnp.concatenate([p, jnp.zeros((_N, 125), jnp.float32)], axis=1)
    xq, xk, xv = _qkv(x, Wq, r(bq), Wk, r(bk), Wv, r(bv))
    tab = jnp.concatenate([xk, xv, p128], axis=1)
    idx = _knn(p, p.T)
    gc = _sc_gather(tab, idx.reshape(-1))
    gc3 = gc.reshape(_N, _S16, _TW)
    hst = _hstats(gc3, p, Wp1, r(bp1))
    ws = _wstats(gc3, p, xq, Wp1, r(bp1), r(gp), r(bpn), Wp2, r(bp2), hst)
    s31, s32, s3st = _s3_stage(gc3, p, xq, Wp1, r(bp1), r(gp), r(bpn),
                               Wp2, r(bp2), hst, ws, r(gw1), Ww1, r(bw1))
    x1, x2 = _final_stage(gc3, p, s31, s32, Wp1, r(bp1), r(gp), r(bpn),
                          Wp2, r(bp2), hst, s3st, r(gw2), Ww2, r(bw2))
    return jnp.concatenate([x1, x2], axis=-1)
